# baseline (speedup 1.0000x reference)
import functools

import jax
import jax.numpy as jnp
from jax.experimental import pallas as pl
from jax.experimental.pallas import tpu as pltpu

NUM_EXPERTS = 8
TOP_K = 2
D_MODEL = 768
D_FF = 2048
SEQ = 2048

BT = 256
BF = 512


def _router_body(x_ref, wr_ref, cmb_ref):
    logits = jnp.dot(x_ref[...], wr_ref[...], preferred_element_type=jnp.float32)
    lane = jax.lax.broadcasted_iota(jnp.int32, logits.shape, 1)
    i1 = jnp.argmax(logits, axis=1, keepdims=True)
    oh1 = lane == i1
    m1 = jnp.max(logits, axis=1, keepdims=True)
    masked = jnp.where(oh1, -jnp.inf, logits)
    i2 = jnp.argmax(masked, axis=1, keepdims=True)
    oh2 = lane == i2
    m2 = jnp.max(masked, axis=1, keepdims=True)
    w1 = 1.0 / (1.0 + jnp.exp(m2 - m1))
    w2 = 1.0 / (1.0 + jnp.exp(m1 - m2))
    cmb_ref[...] = jnp.where(oh1, w1, 0.0) + jnp.where(oh2, w2, 0.0)


def _moe_body(x_ref, cmb_ref, w1_ref, w3_ref, w2_ref, out_ref):
    e = pl.program_id(1)
    f = pl.program_id(2)

    @pl.when(jnp.logical_and(e == 0, f == 0))
    def _():
        out_ref[...] = jnp.zeros_like(out_ref)

    x = x_ref[...]
    a = jnp.dot(x, w1_ref[0], preferred_element_type=jnp.float32)
    b = jnp.dot(x, w3_ref[0], preferred_element_type=jnp.float32)
    h = (a / (1.0 + jnp.exp(-a))) * b
    part = jnp.dot(h, w2_ref[0], preferred_element_type=jnp.float32)
    lane = jax.lax.broadcasted_iota(jnp.int32, cmb_ref.shape, 1)
    cmb_col = jnp.sum(jnp.where(lane == e, cmb_ref[...], 0.0), axis=1,
                      keepdims=True)
    out_ref[...] += cmb_col * part


@functools.partial(jax.jit, static_argnames=())
def _run(flat, Wr, W1, W2, W3):
    T = flat.shape[0]
    combine = pl.pallas_call(
        _router_body,
        out_shape=jax.ShapeDtypeStruct((T, NUM_EXPERTS), jnp.float32),
    )(flat, Wr)

    nt = T // BT
    nf = D_FF // BF
    out = pl.pallas_call(
        _moe_body,
        grid=(nt, NUM_EXPERTS, nf),
        in_specs=[
            pl.BlockSpec((BT, D_MODEL), lambda t, e, f: (t, 0)),
            pl.BlockSpec((BT, NUM_EXPERTS), lambda t, e, f: (t, 0)),
            pl.BlockSpec((1, D_MODEL, BF), lambda t, e, f: (e, 0, f)),
            pl.BlockSpec((1, D_MODEL, BF), lambda t, e, f: (e, 0, f)),
            pl.BlockSpec((1, BF, D_MODEL), lambda t, e, f: (e, f, 0)),
        ],
        out_specs=pl.BlockSpec((BT, D_MODEL), lambda t, e, f: (t, 0)),
        out_shape=jax.ShapeDtypeStruct((T, D_MODEL), jnp.float32),
    )(flat, combine, W1, W3, W2)
    return out


def kernel(hidden_states, Wr, W1, W2, W3):
    batch, seq, hidden = hidden_states.shape
    flat = hidden_states.reshape(-1, hidden)
    out = _run(flat, Wr, W1, W2, W3)
    return out.reshape(batch, seq, hidden)


# speedup vs baseline: 2.4930x; 2.4930x over previous
import functools

import jax
import jax.numpy as jnp
from jax import lax
from jax.experimental import pallas as pl
from jax.experimental.pallas import tpu as pltpu
from jax.experimental.pallas import tpu_sc as plsc

NUM_EXPERTS = 8
TOP_K = 2
D_MODEL = 768
D_FF = 2048
SEQ = 2048

BT = 128
NPAIR = SEQ * TOP_K
NPAD = 5120
NBLK = NPAD // BT
NW = 32
PPW = NPAIR // NW
TPW = SEQ // NW

def _sc_mesh():
    return plsc.VectorSubcoreMesh(
        core_axis_name="c", subcore_axis_name="s", num_cores=2, num_subcores=16)


def _router_body(x_ref, wr_ref, rwb_ref, slots_ref, cnts_ref):
    logits = jnp.dot(x_ref[...], wr_ref[...], preferred_element_type=jnp.float32)
    T = logits.shape[0]
    lane = lax.broadcasted_iota(jnp.int32, logits.shape, 1)
    i1 = jnp.argmax(logits, axis=1, keepdims=True)
    oh1 = lane == i1
    m1 = jnp.max(logits, axis=1, keepdims=True)
    masked = jnp.where(oh1, -jnp.inf, logits)
    i2 = jnp.argmax(masked, axis=1, keepdims=True)
    oh2 = lane == i2
    m2 = jnp.max(masked, axis=1, keepdims=True)
    w1 = 1.0 / (1.0 + jnp.exp(m2 - m1))
    w2 = 1.0 / (1.0 + jnp.exp(m1 - m2))

    C = oh1.astype(jnp.float32) + oh2.astype(jnp.float32)
    r_i = lax.broadcasted_iota(jnp.int32, (T, T), 0)
    c_i = lax.broadcasted_iota(jnp.int32, (T, T), 1)
    tri = (c_i < r_i).astype(jnp.float32)
    S = jnp.dot(tri, C, preferred_element_type=jnp.float32)

    cnts = jnp.sum(C, axis=0, keepdims=True)
    padded = jnp.ceil(cnts / BT) * BT
    a_i = lax.broadcasted_iota(jnp.int32, (NUM_EXPERTS, NUM_EXPERTS), 0)
    b_i = lax.broadcasted_iota(jnp.int32, (NUM_EXPERTS, NUM_EXPERTS), 1)
    upper = (a_i < b_i).astype(jnp.float32)
    start = jnp.dot(padded, upper, preferred_element_type=jnp.float32)

    s1 = jnp.sum(jnp.where(oh1, S, 0.0), axis=1, keepdims=True)
    s2 = jnp.sum(jnp.where(oh2, S, 0.0), axis=1, keepdims=True)
    st1 = jnp.sum(jnp.where(oh1, start, 0.0), axis=1, keepdims=True)
    st2 = jnp.sum(jnp.where(oh2, start, 0.0), axis=1, keepdims=True)
    slot1 = (s1 + st1).astype(jnp.int32)
    slot2 = (s2 + st2).astype(jnp.int32)

    slots_ref[...] = jnp.concatenate([slot1, slot2], axis=1)
    rwb_ref[...] = jnp.concatenate(
        [jnp.broadcast_to(w1, (T, 128)), jnp.broadcast_to(w2, (T, 128))], axis=1)
    cnts_ref[...] = cnts


def _dispatch_body(flat_hbm, slots_hbm, rwb_hbm, xs_hbm, wgt_hbm,
                   tok_v, slot_v, rows_v, wrows_v, sem):
    wid = lax.axis_index("s") * 2 + lax.axis_index("c")
    base = wid * PPW
    pltpu.sync_copy(slots_hbm.at[pl.ds(base, PPW)], slot_v.at[0])
    for g in range(PPW // 16):
        vals = lax.iota(jnp.int32, 16) + (base + 16 * g)
        tok_v[0, pl.ds(16 * g, 16)] = lax.shift_right_logical(vals, 1)
    pltpu.async_copy(flat_hbm.at[tok_v.at[0]], rows_v, sem).wait()
    pltpu.async_copy(rows_v, xs_hbm.at[slot_v.at[0]], sem).wait()
    pltpu.sync_copy(rwb_hbm.at[pl.ds(base, PPW)], wrows_v)
    pltpu.async_copy(wrows_v, wgt_hbm.at[slot_v.at[0]], sem).wait()


def _moe_body(be_ref, xs_ref, w1_ref, w3_ref, w2_ref, wgt_ref, out_ref):
    x = xs_ref[...]
    a = jnp.dot(x, w1_ref[0], preferred_element_type=jnp.float32)
    b = jnp.dot(x, w3_ref[0], preferred_element_type=jnp.float32)
    h = (a / (1.0 + jnp.exp(-a))) * b
    y = jnp.dot(h, w2_ref[0], preferred_element_type=jnp.float32)
    out_ref[...] = y * wgt_ref[...][:, 0:1]


def _combine_body(yw_hbm, slots_hbm, out_hbm, slot_v, yw_v, out_v, sem):
    wid = lax.axis_index("s") * 2 + lax.axis_index("c")
    for c in range(2):
        basep = wid * PPW + 64 * c
        pltpu.sync_copy(slots_hbm.at[pl.ds(basep, 64)], slot_v.at[0])
        pltpu.async_copy(yw_hbm.at[slot_v.at[0]], yw_v, sem).wait()

        def body_i(i, carry):
            def body_j(j, carry2):
                s = pl.ds(j * 16, 16)
                out_v[i, s] = yw_v[2 * i, s] + yw_v[2 * i + 1, s]
                return carry2
            return lax.fori_loop(0, D_MODEL // 16, body_j, carry)

        lax.fori_loop(0, 32, body_i, 0)
        pltpu.sync_copy(out_v, out_hbm.at[pl.ds(wid * TPW + 32 * c, 32)])


def _dispatch_call(flat, slots_flat, rwb):
    fn = pl.kernel(
        _dispatch_body,
        out_type=[jax.ShapeDtypeStruct((NPAD, D_MODEL), jnp.float32),
                  jax.ShapeDtypeStruct((NPAD, 128), jnp.float32)],
        mesh=_sc_mesh(),
        scratch_types=[pltpu.VMEM((1, PPW), jnp.int32),
                       pltpu.VMEM((1, PPW), jnp.int32),
                       pltpu.VMEM((PPW, D_MODEL), jnp.float32),
                       pltpu.VMEM((PPW, 128), jnp.float32),
                       pltpu.SemaphoreType.DMA],
    )
    return fn(flat, slots_flat, rwb)


def _combine_call(yw, slots_flat):
    fn = pl.kernel(
        _combine_body,
        out_type=jax.ShapeDtypeStruct((SEQ, D_MODEL), jnp.float32),
        mesh=_sc_mesh(),
        scratch_types=[pltpu.VMEM((1, 64), jnp.int32),
                       pltpu.VMEM((64, D_MODEL), jnp.float32),
                       pltpu.VMEM((32, D_MODEL), jnp.float32),
                       pltpu.SemaphoreType.DMA],
    )
    return fn(yw, slots_flat)


def kernel(hidden_states, Wr, W1, W2, W3):
    batch, seq, hidden = hidden_states.shape
    flat = hidden_states.reshape(-1, hidden)

    rwb2, slots2, cnts = pl.pallas_call(
        _router_body,
        out_shape=[jax.ShapeDtypeStruct((SEQ, 2 * 128), jnp.float32),
                   jax.ShapeDtypeStruct((SEQ, TOP_K), jnp.int32),
                   jax.ShapeDtypeStruct((1, NUM_EXPERTS), jnp.float32)],
    )(flat, Wr)

    slots_flat = slots2.reshape(NPAIR)
    rwb = rwb2.reshape(NPAIR, 128)

    padded = (jnp.ceil(cnts[0] / BT) * BT).astype(jnp.int32)
    pstart = (jnp.concatenate([jnp.zeros((1,), jnp.int32),
                               jnp.cumsum(padded)[:-1]]) // BT)
    be = jnp.clip(
        (jnp.arange(NBLK, dtype=jnp.int32)[:, None] >= pstart[None, :])
        .sum(axis=1).astype(jnp.int32) - 1, 0, NUM_EXPERTS - 1)

    xs, wgt = _dispatch_call(flat, slots_flat, rwb)

    yw = pl.pallas_call(
        _moe_body,
        grid_spec=pltpu.PrefetchScalarGridSpec(
            num_scalar_prefetch=1,
            grid=(NBLK,),
            in_specs=[
                pl.BlockSpec((BT, D_MODEL), lambda i, be_r: (i, 0)),
                pl.BlockSpec((1, D_MODEL, D_FF), lambda i, be_r: (be_r[i], 0, 0)),
                pl.BlockSpec((1, D_MODEL, D_FF), lambda i, be_r: (be_r[i], 0, 0)),
                pl.BlockSpec((1, D_FF, D_MODEL), lambda i, be_r: (be_r[i], 0, 0)),
                pl.BlockSpec((BT, 128), lambda i, be_r: (i, 0)),
            ],
            out_specs=pl.BlockSpec((BT, D_MODEL), lambda i, be_r: (i, 0)),
        ),
        out_shape=jax.ShapeDtypeStruct((NPAD, D_MODEL), jnp.float32),
    )(be, xs, W1, W3, W2, wgt)

    out = _combine_call(yw, slots_flat)
    return out.reshape(batch, seq, hidden)
